# Initial kernel scaffold; baseline (speedup 1.0000x reference)
import jax
import jax.numpy as jnp
from jax.experimental import pallas as pl


def kernel(x, edge_index, edge_weight, edge_attr, mlp_w1, mlp_b1, mlp_w2, mlp_b2, wq, bq, wk, bk, wv, bv, we, be, wskip, bskip, lin_w, lin_b, out_w, out_b):
    raise NotImplementedError("write your pallas kernel here")


# speedup vs baseline: 2.1258x; 2.1258x over previous
import functools
import math

import jax
import jax.numpy as jnp
from jax.experimental import pallas as pl

N = 10000
E = 160000
HID = 128
HEADS = 8
DH = HID // HEADS
G = 50
FIL = 128
L = 2
CUTOFF = 10.0
SHIFT = float(math.log(2.0))
INV_SQRT_DH = 1.0 / math.sqrt(float(DH))

TILE_E = 1280
TILE_N = 1000


def _ssp(v):
    return jax.nn.softplus(v) - SHIFT


def _edge_body(ea_ref, ew_ref, w1_ref, b1_ref, w2_ref, b2_ref, we_ref, be_ref,
               out_ref):
    t = jnp.dot(ea_ref[...], w1_ref[...], preferred_element_type=jnp.float32)
    t = _ssp(t + b1_ref[...])
    e = jnp.dot(t, w2_ref[...], preferred_element_type=jnp.float32) + b2_ref[...]
    c = 0.5 * (jnp.cos(ew_ref[...] * (math.pi / CUTOFF)) + 1.0)
    e = e * c
    out_ref[...] = (jnp.dot(e, we_ref[...], preferred_element_type=jnp.float32)
                    + be_ref[...])


def _edge_proj(edge_attr, ew2d, w1, b1, w2, b2, we, be):
    grid = (E // TILE_E,)
    return pl.pallas_call(
        _edge_body,
        grid=grid,
        in_specs=[
            pl.BlockSpec((TILE_E, G), lambda i: (i, 0)),
            pl.BlockSpec((TILE_E, 1), lambda i: (i, 0)),
            pl.BlockSpec((G, FIL), lambda i: (0, 0)),
            pl.BlockSpec((1, FIL), lambda i: (0, 0)),
            pl.BlockSpec((FIL, FIL), lambda i: (0, 0)),
            pl.BlockSpec((1, FIL), lambda i: (0, 0)),
            pl.BlockSpec((FIL, HID), lambda i: (0, 0)),
            pl.BlockSpec((1, HID), lambda i: (0, 0)),
        ],
        out_specs=pl.BlockSpec((TILE_E, HID), lambda i: (i, 0)),
        out_shape=jax.ShapeDtypeStruct((E, HID), jnp.float32),
    )(edge_attr, ew2d, w1, b1, w2, b2, we, be)


def _node_body(h_ref, w_ref, b_ref, out_ref):
    out_ref[...] = (jnp.dot(h_ref[...], w_ref[...],
                            preferred_element_type=jnp.float32) + b_ref[...])


def _node_proj(h, wcat, bcat):
    k = wcat.shape[1]
    grid = (N // TILE_N,)
    return pl.pallas_call(
        _node_body,
        grid=grid,
        in_specs=[
            pl.BlockSpec((TILE_N, HID), lambda i: (i, 0)),
            pl.BlockSpec((HID, k), lambda i: (0, 0)),
            pl.BlockSpec((1, k), lambda i: (0, 0)),
        ],
        out_specs=pl.BlockSpec((TILE_N, k), lambda i: (i, 0)),
        out_shape=jax.ShapeDtypeStruct((N, k), jnp.float32),
    )(h, wcat, bcat)


def _post_body(aggnum_ref, denomr_ref, skip_ref, h_ref, lw_ref, lb_ref,
               out_ref):
    agg = aggnum_ref[...] / (denomr_ref[...] + 1e-16)
    t = _ssp(agg + skip_ref[...])
    out_ref[...] = (h_ref[...]
                    + jnp.dot(t, lw_ref[...],
                              preferred_element_type=jnp.float32)
                    + lb_ref[...])


def _post(aggnum, denomrep, skip, h, lin_w, lin_b):
    grid = (N // TILE_N,)
    return pl.pallas_call(
        _post_body,
        grid=grid,
        in_specs=[
            pl.BlockSpec((TILE_N, HID), lambda i: (i, 0)),
            pl.BlockSpec((TILE_N, HID), lambda i: (i, 0)),
            pl.BlockSpec((TILE_N, HID), lambda i: (i, 0)),
            pl.BlockSpec((TILE_N, HID), lambda i: (i, 0)),
            pl.BlockSpec((HID, HID), lambda i: (0, 0)),
            pl.BlockSpec((1, HID), lambda i: (0, 0)),
        ],
        out_specs=pl.BlockSpec((TILE_N, HID), lambda i: (i, 0)),
        out_shape=jax.ShapeDtypeStruct((N, HID), jnp.float32),
    )(aggnum, denomrep, skip, h, lin_w, lin_b)


def _attention(q, k, v, ep, src, dst):
    kj = k[src] + ep
    vj = v[src] + ep
    qi = q[dst]
    logits = (qi.reshape(E, HEADS, DH) * kj.reshape(E, HEADS, DH)).sum(-1)
    ea = jnp.exp(logits * INV_SQRT_DH)
    denom = jax.ops.segment_sum(ea, dst, num_segments=N)
    earep = jnp.repeat(ea, DH, axis=1)
    aggnum = jax.ops.segment_sum(vj * earep, dst, num_segments=N)
    return aggnum, denom


def kernel(x, edge_index, edge_weight, edge_attr, mlp_w1, mlp_b1, mlp_w2,
           mlp_b2, wq, bq, wk, bk, wv, bv, we, be, wskip, bskip, lin_w, lin_b,
           out_w, out_b):
    src = edge_index[0]
    dst = edge_index[1]
    ew2d = edge_weight.reshape(E, 1)
    h = x
    for l in range(L):
        ep = _edge_proj(edge_attr, ew2d, mlp_w1[l], mlp_b1[l].reshape(1, FIL),
                        mlp_w2[l], mlp_b2[l].reshape(1, FIL), we[l],
                        be[l].reshape(1, HID))
        wcat = jnp.concatenate([wq[l], wk[l], wv[l], wskip[l]], axis=1)
        bcat = jnp.concatenate([bq[l], bk[l], bv[l], bskip[l]]).reshape(1, 4 * HID)
        qkvs = _node_proj(h, wcat, bcat)
        q = qkvs[:, 0:HID]
        k = qkvs[:, HID:2 * HID]
        v = qkvs[:, 2 * HID:3 * HID]
        skip = qkvs[:, 3 * HID:4 * HID]
        aggnum, denom = _attention(q, k, v, ep, src, dst)
        denomrep = jnp.repeat(denom, DH, axis=1)
        h = _post(aggnum, denomrep, skip, h, lin_w[l],
                  lin_b[l].reshape(1, HID))
    return _node_proj(h, out_w, out_b.reshape(1, HID))
